# baseline (speedup 1.0000x reference)
import functools

import jax
import jax.numpy as jnp
from jax import lax
from jax.experimental import pallas as pl
from jax.experimental.pallas import tpu as pltpu
from jax.experimental.pallas import tpu_sc as plsc

NC = 2
NS = 16
NW = NC * NS
RPB = 128
NBUF = 2
TW = 256
TSTRIPE = 31232


def _pair_fn(v, d):
    mesh = plsc.VectorSubcoreMesh(
        core_axis_name="c", subcore_axis_name="s",
        num_cores=NC, num_subcores=NS,
    )
    dd = 2 * d
    tail0 = v - (v % (2 * TW))
    ntail = (v - tail0) // 2

    @functools.partial(
        pl.kernel,
        out_type=jax.ShapeDtypeStruct((v // 2, dd), jnp.float32),
        mesh=mesh,
        compiler_params=pltpu.CompilerParams(needs_layout_passes=False),
        scratch_types=[
            pltpu.VMEM((NBUF, d, TW), jnp.float32),
            pltpu.VMEM((NBUF, TW // 2, dd), jnp.float32),
            pltpu.SemaphoreType.DMA,
            pltpu.SemaphoreType.DMA,
        ],
    )
    def pair_kernel(tt_hbm, tail2_hbm, t2_hbm, slab, tbuf, gsem, ssem):
        wid = lax.axis_index("s") * NC + lax.axis_index("c")
        c0 = wid * TSTRIPE
        nch = TSTRIPE // TW + jnp.where(wid == NW - 1, 2, 0)

        rows4 = [jax.lax.iota(jnp.int32, 16) + jnp.int32(16 * r)
                 for r in range(d // 16)]

        def transpose_chunk(b):
            @plsc.parallel_loop(0, TW // 2, unroll=8)
            def _(q):
                cols = [jnp.full((16,), 2 * q, jnp.int32)]
                cols.append(cols[0] + 1)
                for m in range(dd // 16):
                    tbuf[b, q, pl.ds(16 * m, 16)] = plsc.load_gather(
                        slab.at[b], [rows4[m % 4], cols[m // 4]])

        for b in range(NBUF):
            pltpu.async_copy(
                tt_hbm.at[:, pl.ds(c0 + b * TW, TW)], slab.at[b], gsem)

        @pl.loop(0, (TSTRIPE // TW + 2) // NBUF)
        def _(g):
            for b in range(NBUF):
                ch = g * NBUF + b

                @pl.when(ch < nch)
                def _():
                    pltpu.make_async_copy(
                        tt_hbm.at[:, pl.ds(0, TW)], slab.at[b], gsem).wait()

                    @pl.when(ch >= NBUF)
                    def _():
                        pltpu.make_async_copy(
                            tbuf.at[b], t2_hbm.at[pl.ds(0, TW // 2)], ssem
                        ).wait()

                    transpose_chunk(b)
                    q0 = pl.multiple_of((c0 + ch * TW) // 2, TW // 2)
                    pltpu.async_copy(
                        tbuf.at[b], t2_hbm.at[pl.ds(q0, TW // 2)], ssem)

                    @pl.when(ch + NBUF < nch)
                    def _():
                        cnx = pl.multiple_of(c0 + (ch + NBUF) * TW, TW)
                        pltpu.async_copy(
                            tt_hbm.at[:, pl.ds(cnx, TW)], slab.at[b], gsem)

        for b in range(NBUF):
            pltpu.make_async_copy(
                tbuf.at[b], t2_hbm.at[pl.ds(0, TW // 2)], ssem).wait()

        @pl.when(wid == 0)
        def _():
            pltpu.sync_copy(tail2_hbm, tbuf.at[0, pl.ds(0, ntail)])
            pltpu.sync_copy(tbuf.at[0, pl.ds(0, ntail)],
                            t2_hbm.at[pl.ds(tail0 // 2, ntail)])

    return pair_kernel


def _gather_fn(batch, seq, v, d):
    mesh = plsc.VectorSubcoreMesh(
        core_axis_name="c", subcore_axis_name="s",
        num_cores=NC, num_subcores=NS,
    )
    dd = 2 * d

    @functools.partial(
        pl.kernel,
        out_type=jax.ShapeDtypeStruct((seq, d, batch), jnp.float32),
        mesh=mesh,
        compiler_params=pltpu.CompilerParams(needs_layout_passes=False),
        scratch_types=[
            pltpu.VMEM((seq, RPB), jnp.int32),
            pltpu.VMEM((seq, RPB), jnp.int32),
            pltpu.VMEM((NBUF, RPB, dd), jnp.float32),
            pltpu.VMEM((NBUF, d, RPB), jnp.float32),
            pltpu.SemaphoreType.DMA,
            pltpu.SemaphoreType.DMA,
        ],
    )
    def gather_kernel(ids_hbm, table_hbm, out_hbm, idx_v, idx2_v,
                      gbuf, tbuf, gsem, ssem):
        wid = lax.axis_index("s") * NC + lax.axis_index("c")
        b0 = wid * RPB
        pltpu.sync_copy(ids_hbm.at[:, pl.ds(b0, RPB)], idx_v)

        @pl.loop(0, seq)
        def _(j):
            for k in range(RPB // 16):
                vv = idx_v[j, pl.ds(16 * k, 16)]
                idx2_v[j, pl.ds(16 * k, 16)] = lax.shift_right_logical(vv, 1)

        for b in range(NBUF):
            pltpu.async_copy(table_hbm.at[idx2_v.at[b]], gbuf.at[b], gsem)

        @pl.loop(0, seq, step=NBUF)
        def _(g):
            for b in range(NBUF):
                j = g + b
                pltpu.make_async_copy(
                    table_hbm.at[idx2_v.at[0]], gbuf.at[b], gsem
                ).wait()

                @pl.when(j >= NBUF)
                def _():
                    pltpu.make_async_copy(
                        tbuf.at[b], out_hbm.at[0, :, pl.ds(0, RPB)], ssem
                    ).wait()

                for k in range(RPB // 16):
                    vv = idx_v[j, pl.ds(16 * k, 16)]
                    off = lax.shift_left(
                        lax.bitwise_and(vv, jnp.int32(1)), 6)
                    rows = jax.lax.iota(jnp.int32, 16) + jnp.int32(16 * k)

                    @plsc.parallel_loop(0, d, unroll=16)
                    def _(di):
                        vals = plsc.load_gather(gbuf.at[b], [rows, off + di])
                        tbuf[b, di, pl.ds(16 * k, 16)] = vals

                pltpu.async_copy(
                    tbuf.at[b], out_hbm.at[j, :, pl.ds(b0, RPB)], ssem
                )
                @pl.when(j + NBUF < seq)
                def _():
                    pltpu.async_copy(
                        table_hbm.at[idx2_v.at[j + NBUF]], gbuf.at[b], gsem
                    )

        for b in range(NBUF):
            pltpu.make_async_copy(
                tbuf.at[b], out_hbm.at[0, :, pl.ds(0, RPB)], ssem
            ).wait()

    return gather_kernel


def kernel(input_ids, table):
    batch, seq = input_ids.shape
    v, d = table.shape
    assert batch == NW * RPB and seq % NBUF == 0 and v % 2 == 0
    assert d == 64 and v == NW * TSTRIPE + 2 * TW + 64
    ids_t = input_ids.T
    tt = table.T
    tail0 = v - (v % (2 * TW))
    tail2 = table[tail0:, :].reshape((v - tail0) // 2, 2 * d)
    table2 = _pair_fn(v, d)(tt, tail2)
    out_t = _gather_fn(batch, seq, v, d)(ids_t, table2)
    return jnp.transpose(out_t, (2, 0, 1))


# speedup vs baseline: 1.2650x; 1.2650x over previous
import functools

import jax
import jax.numpy as jnp
from jax import lax
from jax.experimental import pallas as pl
from jax.experimental.pallas import tpu as pltpu
from jax.experimental.pallas import tpu_sc as plsc

NC = 2
NS = 16
NW = NC * NS
RPB = 128
NBUF = 2
GP = 4


def _gather_fn(batch, seq, v, d):
    mesh = plsc.VectorSubcoreMesh(
        core_axis_name="c", subcore_axis_name="s",
        num_cores=NC, num_subcores=NS,
    )
    dd = 2 * d

    @functools.partial(
        pl.kernel,
        out_type=jax.ShapeDtypeStruct((seq, d, batch), jnp.float32),
        mesh=mesh,
        compiler_params=pltpu.CompilerParams(needs_layout_passes=False),
        scratch_types=[
            pltpu.VMEM((seq, RPB), jnp.int32),
            pltpu.VMEM((seq, RPB), jnp.int32),
            pltpu.VMEM((NBUF, RPB, dd + GP), jnp.float32),
            pltpu.VMEM((NBUF, d, RPB), jnp.float32),
            pltpu.SemaphoreType.DMA,
            pltpu.SemaphoreType.DMA,
        ],
    )
    def gather_kernel(ids_hbm, table_hbm, out_hbm, idx_v, idx2_v,
                      gbuf, tbuf, gsem, ssem):
        wid = lax.axis_index("s") * NC + lax.axis_index("c")
        b0 = wid * RPB
        pltpu.sync_copy(ids_hbm.at[:, pl.ds(b0, RPB)], idx_v)

        @pl.loop(0, seq)
        def _(j):
            for k in range(RPB // 16):
                vv = idx_v[j, pl.ds(16 * k, 16)]
                idx2_v[j, pl.ds(16 * k, 16)] = lax.shift_right_logical(vv, 1)

        def gdst(b):
            return gbuf.at[b, :, pl.ds(0, dd)]

        for b in range(NBUF):
            pltpu.async_copy(table_hbm.at[idx2_v.at[b]], gdst(b), gsem)

        @pl.loop(0, seq, step=NBUF)
        def _(g):
            for b in range(NBUF):
                j = g + b
                pltpu.make_async_copy(
                    table_hbm.at[idx2_v.at[0]], gdst(b), gsem
                ).wait()

                @pl.when(j >= NBUF)
                def _():
                    pltpu.make_async_copy(
                        tbuf.at[b], out_hbm.at[0, :, pl.ds(0, RPB)], ssem
                    ).wait()

                for k in range(RPB // 16):
                    vv = idx_v[j, pl.ds(16 * k, 16)]
                    off = lax.shift_left(
                        lax.bitwise_and(vv, jnp.int32(1)), 6)
                    rows = jax.lax.iota(jnp.int32, 16) + jnp.int32(16 * k)

                    @plsc.parallel_loop(0, d, unroll=16)
                    def _(di):
                        vals = plsc.load_gather(gbuf.at[b], [rows, off + di])
                        tbuf[b, di, pl.ds(16 * k, 16)] = vals

                pltpu.async_copy(
                    tbuf.at[b], out_hbm.at[j, :, pl.ds(b0, RPB)], ssem
                )
                @pl.when(j + NBUF < seq)
                def _():
                    pltpu.async_copy(
                        table_hbm.at[idx2_v.at[j + NBUF]], gdst(b), gsem
                    )

        for b in range(NBUF):
            pltpu.make_async_copy(
                tbuf.at[b], out_hbm.at[0, :, pl.ds(0, RPB)], ssem
            ).wait()

    return gather_kernel


def kernel(input_ids, table):
    batch, seq = input_ids.shape
    v, d = table.shape
    assert batch == NW * RPB and seq % NBUF == 0 and v % 2 == 0 and d == 64
    ids_t = input_ids.T
    table2 = table.reshape(v // 2, 2 * d)
    out_t = _gather_fn(batch, seq, v, d)(ids_t, table2)
    return jnp.transpose(out_t, (2, 0, 1))


# speedup vs baseline: 1.2786x; 1.0108x over previous
import functools

import jax
import jax.numpy as jnp
from jax import lax
from jax.experimental import pallas as pl
from jax.experimental.pallas import tpu as pltpu
from jax.experimental.pallas import tpu_sc as plsc

NC = 2
NS = 16
NW = NC * NS
RPB = 128
NBUF = 5


def _gather_fn(nstep, d):
    mesh = plsc.VectorSubcoreMesh(
        core_axis_name="c", subcore_axis_name="s",
        num_cores=NC, num_subcores=NS,
    )

    @functools.partial(
        pl.kernel,
        out_type=jax.ShapeDtypeStruct((NW * nstep * RPB, d), jnp.float32),
        mesh=mesh,
        compiler_params=pltpu.CompilerParams(use_tc_tiling_on_sc=False),
        scratch_types=[
            pltpu.VMEM((nstep, RPB), jnp.int32),
            pltpu.VMEM((NBUF, RPB, d), jnp.float32),
            pltpu.SemaphoreType.DMA,
            pltpu.SemaphoreType.DMA,
        ],
    )
    def gather_kernel(ids_hbm, table_hbm, out_hbm, idx_v, bufs, gsem, ssem):
        wid = lax.axis_index("s") * NC + lax.axis_index("c")
        base = wid * nstep * RPB
        pltpu.sync_copy(ids_hbm.at[wid], idx_v)

        for b in range(NBUF):
            pltpu.async_copy(table_hbm.at[idx_v.at[b]], bufs.at[b], gsem)

        @pl.loop(0, nstep, step=NBUF)
        def _(g):
            for b in range(NBUF):
                j = g + b
                pltpu.make_async_copy(
                    table_hbm.at[idx_v.at[0]], bufs.at[b], gsem
                ).wait()
                st = pltpu.async_copy(
                    bufs.at[b], out_hbm.at[pl.ds(base + j * RPB, RPB)], ssem
                )
                st.wait()
                @pl.when(j + NBUF < nstep)
                def _():
                    pltpu.async_copy(
                        table_hbm.at[idx_v.at[j + NBUF]], bufs.at[b], gsem
                    )

    return gather_kernel


def kernel(input_ids, table):
    batch, seq = input_ids.shape
    _, d = table.shape
    n = batch * seq
    assert n % (NW * RPB) == 0
    nstep = n // (NW * RPB)
    assert nstep % NBUF == 0
    ids = input_ids.reshape(NW, nstep, RPB)
    out = _gather_fn(nstep, d)(ids, table)
    return out.reshape(batch, seq, d)
